# baseline (speedup 1.0000x reference)
import functools

import jax
import jax.numpy as jnp
from jax import lax
from jax.experimental import pallas as pl
from jax.experimental.pallas import tpu as pltpu
from jax.experimental.pallas import tpu_sc as plsc

N = 10000
F = 128
H = 128
C = 64
E = 320000

N_PAD = 10240
CHUNK = 128
NW = 32
CPW = 80
E_PAD = NW * CHUNK * CPW
IGRP = 16
DW = 16
ROWS_PT = N_PAD // 16


def _zero_spmem(zb, acc_sh, base, W):
    def zb_row(i, _):
        def zb_col(k, _2):
            zb[i, pl.ds(k * 16, 16)] = jnp.zeros((16,), jnp.float32)
            return 0
        return lax.fori_loop(0, W // 16, zb_col, 0)
    lax.fori_loop(0, 8, zb_row, 0)

    def zero_acc(g, _):
        pltpu.sync_copy(zb, acc_sh.at[pl.ds(base + g * 8, 8)])
        return 0
    lax.fori_loop(0, ROWS_PT // 8, zero_acc, 0)


def _make_sc_agg(W, with_deg=False):
    mesh = plsc.VectorSubcoreMesh(core_axis_name="c", subcore_axis_name="s")

    out_type = [jax.ShapeDtypeStruct((2, N_PAD, W), jnp.float32)]
    scratch_types = [
        pltpu.VMEM((IGRP, CHUNK), jnp.int32),
        pltpu.VMEM((IGRP, CHUNK), jnp.int32),
        pltpu.VMEM((CHUNK, W), jnp.float32),
        pltpu.VMEM((CHUNK, W), jnp.float32),
        pltpu.VMEM((8, W), jnp.float32),
        pltpu.VMEM_SHARED((N_PAD, W), jnp.float32),
        pltpu.SemaphoreType.DMA,
        pltpu.SemaphoreType.DMA,
        pltpu.SemaphoreType.DMA,
        pltpu.SemaphoreType.DMA,
    ]
    if with_deg:
        out_type.append(jax.ShapeDtypeStruct((32, N_PAD), jnp.float32))
        scratch_types.append(pltpu.VMEM((N_PAD,), jnp.float32))

    def body(*refs):
        if with_deg:
            (x_hbm, src_hbm, dst_hbm, parts_out, degp_out,
             src_v, dst_v, rows_a, rows_b, zb, acc_sh,
             sem_a0, sem_a1, sem_b0, sem_b1, hist) = refs
        else:
            (x_hbm, src_hbm, dst_hbm, parts_out,
             src_v, dst_v, rows_a, rows_b, zb, acc_sh,
             sem_a0, sem_a1, sem_b0, sem_b1) = refs
        sem_a = (sem_a0, sem_a1)
        sem_b = (sem_b0, sem_b1)
        c = lax.axis_index("c")
        s = lax.axis_index("s")
        wid = c * 16 + s
        base = s * ROWS_PT
        _zero_spmem(zb, acc_sh, base, W)
        if with_deg:
            def zro(i, _):
                hist[pl.ds(i * 16, 16)] = jnp.zeros((16,), jnp.float32)
                return 0
            lax.fori_loop(0, N_PAD // 16, zro, 0)
        plsc.subcore_barrier()

        row0 = wid * CPW
        bufs = (rows_a, rows_b)
        sems = (sem_a, sem_b)

        def group(g, _):
            pltpu.sync_copy(src_hbm.at[pl.ds(row0 + g * IGRP, IGRP)], src_v)
            pltpu.sync_copy(dst_hbm.at[pl.ds(row0 + g * IGRP, IGRP)], dst_v)
            def start(j):
                b, sm = bufs[j % 2], sems[j % 2]
                return (
                    pltpu.async_copy(x_hbm.at[src_v.at[j, pl.ds(0, 64)]],
                                     b.at[pl.ds(0, 64)], sm[0]),
                    pltpu.async_copy(x_hbm.at[src_v.at[j, pl.ds(64, 64)]],
                                     b.at[pl.ds(64, 64)], sm[1]),
                )
            cp = start(0)
            for j in range(IGRP):
                if with_deg:
                    for k in range(CHUNK // 16):
                        idxv = dst_v[j, pl.ds(k * 16, 16)]
                        plsc.addupdate_scatter(hist, [idxv],
                                               jnp.ones((16,), jnp.float32))
                cp[0].wait()
                cp[1].wait()
                if j + 1 < IGRP:
                    cp = start(j + 1)
                pltpu.sync_copy(bufs[j % 2], acc_sh.at[dst_v.at[j]], add=True)
            return 0
        lax.fori_loop(0, CPW // IGRP, group, 0)

        if with_deg:
            pltpu.sync_copy(hist, degp_out.at[wid])
        plsc.subcore_barrier()
        pltpu.sync_copy(acc_sh.at[pl.ds(base, ROWS_PT)],
                        parts_out.at[c, pl.ds(base, ROWS_PT)])

    return pl.kernel(
        body, mesh=mesh, out_type=out_type, scratch_types=scratch_types,
        compiler_params=pltpu.CompilerParams(needs_layout_passes=False))


_sc_agg = _make_sc_agg(H)
_sc_agg_deg = _make_sc_agg(H, with_deg=True)


R = 1024


def _tc_root(x_ref, wr_ref, b_ref, o_ref):
    o_ref[...] = (jnp.dot(x_ref[...], wr_ref[...],
                          preferred_element_type=jnp.float32) + b_ref[...])


def _tc_mean_combine(relu):
    def body(p_ref, degp_ref, r_ref, wl_ref, o_ref):
        p = p_ref[...]
        agg = p[0] + p[1]
        deg = jnp.sum(degp_ref[...], axis=0)[:, None]
        mean = agg * (1.0 / jnp.maximum(deg, 1.0))
        h = jnp.dot(mean, wl_ref[...],
                    preferred_element_type=jnp.float32) + r_ref[...]
        o_ref[...] = jnp.maximum(h, 0.0) if relu else h
    return body


def _tc_combine_root(p_ref, degp_ref, r_ref, wl_ref, wrn_ref, bn_ref,
                     h_ref, rn_ref):
    p = p_ref[...]
    agg = p[0] + p[1]
    deg = jnp.sum(degp_ref[...], axis=0)[:, None]
    mean = agg * (1.0 / jnp.maximum(deg, 1.0))
    h = jnp.maximum(jnp.dot(mean, wl_ref[...],
                            preferred_element_type=jnp.float32) + r_ref[...],
                    0.0)
    h_ref[...] = h
    rn_ref[...] = (jnp.dot(h, wrn_ref[...],
                           preferred_element_type=jnp.float32) + bn_ref[...])


def _row_spec(w):
    return pl.BlockSpec((R, w), lambda i: (i, 0))


def _part_spec(w):
    return pl.BlockSpec((2, R, w), lambda i: (0, i, 0))


def _deg_spec():
    return pl.BlockSpec((32, R), lambda i: (0, i))


def _full_spec(shape):
    nd = len(shape)
    return pl.BlockSpec(shape, lambda i: (0,) * nd)


def _root_call(x, wrT, b):
    w_out = wrT.shape[1]
    return pl.pallas_call(
        _tc_root,
        grid=(N_PAD // R,),
        in_specs=[_row_spec(H), _full_spec(wrT.shape), _full_spec(b.shape)],
        out_specs=_row_spec(w_out),
        out_shape=jax.ShapeDtypeStruct((N_PAD, w_out), jnp.float32),
    )(x, wrT, b)


def _combine_root_call(p, degp, r, wlT, wrnT, bn):
    wn_out = wrnT.shape[1]
    return pl.pallas_call(
        _tc_combine_root,
        grid=(N_PAD // R,),
        in_specs=[_part_spec(H), _deg_spec(), _row_spec(H),
                  _full_spec(wlT.shape), _full_spec(wrnT.shape),
                  _full_spec(bn.shape)],
        out_specs=[_row_spec(H), _row_spec(wn_out)],
        out_shape=[jax.ShapeDtypeStruct((N_PAD, H), jnp.float32),
                   jax.ShapeDtypeStruct((N_PAD, wn_out), jnp.float32)],
    )(p, degp, r, wlT, wrnT, bn)


def _combine_call(p, degp, r, wlT, relu):
    w_out = wlT.shape[1]
    return pl.pallas_call(
        _tc_mean_combine(relu),
        grid=(N_PAD // R,),
        in_specs=[_part_spec(H), _deg_spec(), _row_spec(w_out),
                  _full_spec(wlT.shape)],
        out_specs=_row_spec(w_out),
        out_shape=jax.ShapeDtypeStruct((N_PAD, w_out), jnp.float32),
    )(p, degp, r, wlT)


def kernel(x, adj, W1l, b1l, W1r, W2l, b2l, W2r, W3l, b3l, W3r):
    x_pad = jnp.zeros((N_PAD, F), jnp.float32).at[:N].set(x)

    src = adj[0]
    dst = adj[1]
    npad = E_PAD - E
    pad_ids = jnp.arange(npad, dtype=jnp.int32)
    src_pad = jnp.concatenate([src, pad_ids % N]).reshape(E_PAD // CHUNK, CHUNK)
    dst_pad = jnp.concatenate([dst, N + pad_ids % (N_PAD - N)]
                              ).reshape(E_PAD // CHUNK, CHUNK)

    r1 = _root_call(x_pad, W1r.T, b1l.reshape(1, H))
    p1, degp = _sc_agg_deg(x_pad, src_pad, dst_pad)
    h1, r2 = _combine_root_call(p1, degp, r1, W1l.T, W2r.T,
                                b2l.reshape(1, H))
    (p2,) = _sc_agg(h1, src_pad, dst_pad)
    h2, r3 = _combine_root_call(p2, degp, r2, W2l.T, W3r.T,
                                b3l.reshape(1, C))
    (p3,) = _sc_agg(h2, src_pad, dst_pad)
    out = _combine_call(p3, degp, r3, W3l.T, relu=False)
    return out[:N]


# speedup vs baseline: 1.0069x; 1.0069x over previous
import functools

import jax
import jax.numpy as jnp
from jax import lax
from jax.experimental import pallas as pl
from jax.experimental.pallas import tpu as pltpu
from jax.experimental.pallas import tpu_sc as plsc

N = 10000
F = 128
H = 128
C = 64
E = 320000

N_PAD = 10240
CHUNK = 128
NW = 32
CPW = 80
E_PAD = NW * CHUNK * CPW
IGRP = 16
DW = 16
ROWS_PT = N_PAD // 16


def _zero_spmem(zb, acc_sh, base, W):
    def zb_row(i, _):
        def zb_col(k, _2):
            zb[i, pl.ds(k * 16, 16)] = jnp.zeros((16,), jnp.float32)
            return 0
        return lax.fori_loop(0, W // 16, zb_col, 0)
    lax.fori_loop(0, 8, zb_row, 0)

    def zero_acc(g, _):
        pltpu.sync_copy(zb, acc_sh.at[pl.ds(base + g * 8, 8)])
        return 0
    lax.fori_loop(0, ROWS_PT // 8, zero_acc, 0)


def _make_sc_agg(W, with_deg=False):
    mesh = plsc.VectorSubcoreMesh(core_axis_name="c", subcore_axis_name="s")

    out_type = [jax.ShapeDtypeStruct((2, N_PAD, W), jnp.float32)]
    scratch_types = [
        pltpu.VMEM((IGRP, CHUNK), jnp.int32),
        pltpu.VMEM((IGRP, CHUNK), jnp.int32),
        pltpu.VMEM((CHUNK, W), jnp.float32),
        pltpu.VMEM((CHUNK, W), jnp.float32),
        pltpu.VMEM((8, W), jnp.float32),
        pltpu.VMEM_SHARED((N_PAD, W), jnp.float32),
        pltpu.SemaphoreType.DMA,
        pltpu.SemaphoreType.DMA,
        pltpu.SemaphoreType.DMA,
        pltpu.SemaphoreType.DMA,
    ]
    if with_deg:
        out_type.append(jax.ShapeDtypeStruct((32, N_PAD), jnp.float32))
        scratch_types.append(pltpu.VMEM((N_PAD,), jnp.float32))

    def body(*refs):
        if with_deg:
            (x_hbm, src_hbm, dst_hbm, parts_out, degp_out,
             src_v, dst_v, rows_a, rows_b, zb, acc_sh,
             sem_a0, sem_a1, sem_b0, sem_b1, hist) = refs
        else:
            (x_hbm, src_hbm, dst_hbm, parts_out,
             src_v, dst_v, rows_a, rows_b, zb, acc_sh,
             sem_a0, sem_a1, sem_b0, sem_b1) = refs
        sem_a = (sem_a0, sem_a1)
        sem_b = (sem_b0, sem_b1)
        c = lax.axis_index("c")
        s = lax.axis_index("s")
        wid = c * 16 + s
        base = s * ROWS_PT
        _zero_spmem(zb, acc_sh, base, W)
        if with_deg:
            def zro(i, _):
                hist[pl.ds(i * 16, 16)] = jnp.zeros((16,), jnp.float32)
                return 0
            lax.fori_loop(0, N_PAD // 16, zro, 0)
        plsc.subcore_barrier()

        row0 = wid * CPW
        bufs = (rows_a, rows_b)
        sems = (sem_a, sem_b)

        def group(g, _):
            pltpu.sync_copy(src_hbm.at[pl.ds(row0 + g * IGRP, IGRP)], src_v)
            pltpu.sync_copy(dst_hbm.at[pl.ds(row0 + g * IGRP, IGRP)], dst_v)
            def start(j):
                b, sm = bufs[j % 2], sems[j % 2]
                return (
                    pltpu.async_copy(x_hbm.at[src_v.at[j, pl.ds(0, 64)]],
                                     b.at[pl.ds(0, 64)], sm[0]),
                    pltpu.async_copy(x_hbm.at[src_v.at[j, pl.ds(64, 64)]],
                                     b.at[pl.ds(64, 64)], sm[1]),
                )
            cp = start(0)
            for j in range(IGRP):
                if with_deg:
                    for k in range(CHUNK // 16):
                        idxv = dst_v[j, pl.ds(k * 16, 16)]
                        plsc.addupdate_scatter(hist, [idxv],
                                               jnp.ones((16,), jnp.float32))
                cp[0].wait()
                cp[1].wait()
                if j + 1 < IGRP:
                    cp = start(j + 1)
                pltpu.sync_copy(bufs[j % 2], acc_sh.at[dst_v.at[j]], add=True)
            return 0
        lax.fori_loop(0, CPW // IGRP, group, 0)

        if with_deg:
            pltpu.sync_copy(hist, degp_out.at[wid])
        plsc.subcore_barrier()
        pltpu.sync_copy(acc_sh.at[pl.ds(base, ROWS_PT)],
                        parts_out.at[c, pl.ds(base, ROWS_PT)])

    return pl.kernel(
        body, mesh=mesh, out_type=out_type, scratch_types=scratch_types,
        compiler_params=pltpu.CompilerParams(needs_layout_passes=False))


_sc_agg = _make_sc_agg(H)
_sc_agg_deg = _make_sc_agg(H, with_deg=True)


R = 1024


def _tc_root(x_ref, wr_ref, b_ref, o_ref):
    o_ref[...] = (jnp.dot(x_ref[...], wr_ref[...],
                          preferred_element_type=jnp.float32) + b_ref[...])


def _tc_mean_combine(relu):
    def body(p_ref, degp_ref, r_ref, wl_ref, o_ref):
        p = p_ref[...]
        agg = p[0] + p[1]
        deg = jnp.sum(degp_ref[...], axis=0)[:, None]
        mean = agg * (1.0 / jnp.maximum(deg, 1.0))
        h = jnp.dot(mean, wl_ref[...],
                    preferred_element_type=jnp.float32) + r_ref[...]
        o_ref[...] = jnp.maximum(h, 0.0) if relu else h
    return body


def _row_spec(w):
    return pl.BlockSpec((R, w), lambda i: (i, 0))


def _part_spec(w):
    return pl.BlockSpec((2, R, w), lambda i: (0, i, 0))


def _deg_spec():
    return pl.BlockSpec((32, R), lambda i: (0, i))


def _full_spec(shape):
    nd = len(shape)
    return pl.BlockSpec(shape, lambda i: (0,) * nd)


def _root_call(x, wrT, b):
    w_out = wrT.shape[1]
    return pl.pallas_call(
        _tc_root,
        grid=(N_PAD // R,),
        in_specs=[_row_spec(H), _full_spec(wrT.shape), _full_spec(b.shape)],
        out_specs=_row_spec(w_out),
        out_shape=jax.ShapeDtypeStruct((N_PAD, w_out), jnp.float32),
    )(x, wrT, b)


def _combine_call(p, degp, r, wlT, relu):
    w_out = wlT.shape[1]
    return pl.pallas_call(
        _tc_mean_combine(relu),
        grid=(N_PAD // R,),
        in_specs=[_part_spec(H), _deg_spec(), _row_spec(w_out),
                  _full_spec(wlT.shape)],
        out_specs=_row_spec(w_out),
        out_shape=jax.ShapeDtypeStruct((N_PAD, w_out), jnp.float32),
    )(p, degp, r, wlT)


def kernel(x, adj, W1l, b1l, W1r, W2l, b2l, W2r, W3l, b3l, W3r):
    x_pad = jnp.zeros((N_PAD, F), jnp.float32).at[:N].set(x)

    src = adj[0]
    dst = adj[1]
    npad = E_PAD - E
    pad_ids = jnp.arange(npad, dtype=jnp.int32)
    src_pad = jnp.concatenate([src, pad_ids % N]).reshape(E_PAD // CHUNK, CHUNK)
    dst_pad = jnp.concatenate([dst, N + pad_ids % (N_PAD - N)]
                              ).reshape(E_PAD // CHUNK, CHUNK)

    r1 = _root_call(x_pad, W1r.T, b1l.reshape(1, H))
    p1, degp = _sc_agg_deg(x_pad, src_pad, dst_pad)
    h1 = _combine_call(p1, degp, r1, W1l.T, relu=True)
    r2 = _root_call(h1, W2r.T, b2l.reshape(1, H))
    (p2,) = _sc_agg(h1, src_pad, dst_pad)
    h2 = _combine_call(p2, degp, r2, W2l.T, relu=True)
    r3 = _root_call(h2, W3r.T, b3l.reshape(1, C))
    (p3,) = _sc_agg(h2, src_pad, dst_pad)
    out = _combine_call(p3, degp, r3, W3l.T, relu=False)
    return out[:N]
